# baseline (speedup 1.0000x reference)
import functools

import jax
import jax.numpy as jnp
from jax import lax
from jax.experimental import pallas as pl
from jax.experimental.pallas import tpu as pltpu
from jax.experimental.pallas import tpu_sc as plsc

_NBINS = 32
_FEATS = 128
_LANES = 16
_NWORKERS = 32
_ROWS_PER_BLOCK = 256
_VECS_PER_ROW = _FEATS // _LANES


def _tokenizer_body(x_hbm, bnd_hbm, out_hbm, xbuf, hist, bnd):
    rows_total = x_hbm.shape[0] // _FEATS
    rows_per_worker = rows_total // _NWORKERS
    n_blocks = rows_per_worker // _ROWS_PER_BLOCK

    wid = lax.axis_index("c") * 16 + lax.axis_index("s")
    row0 = wid * rows_per_worker

    pltpu.sync_copy(bnd_hbm, bnd)

    val16 = jnp.full((_LANES,), 2.0 ** -7, jnp.float32)
    zeros16 = jnp.zeros((_LANES,), jnp.float32)

    def zero_body(i):
        hist[pl.ds(i * _LANES, _LANES)] = zeros16

    def row_body(r):
        rbase = r * _FEATS
        obase = r * _NBINS
        for j in range(_VECS_PER_ROW):
            x16 = xbuf[pl.ds(rbase + j * _LANES, _LANES)]
            t = x16 * jnp.float32(5.0) + jnp.float32(15.5)
            t = jnp.minimum(jnp.maximum(t, jnp.float32(0.0)),
                            jnp.float32(30.0))
            g = t.astype(jnp.int32)
            bg = plsc.load_gather(bnd, [g])
            y = g + (x16 >= bg).astype(jnp.int32)
            plsc.addupdate_scatter(hist, [y + obase], val16)

    def blk_body(blk, _):
        blk_row = row0 + blk * _ROWS_PER_BLOCK
        pltpu.sync_copy(
            x_hbm.at[pl.ds(blk_row * _FEATS, _ROWS_PER_BLOCK * _FEATS)],
            xbuf)
        plsc.parallel_loop(0, _ROWS_PER_BLOCK * _NBINS // _LANES,
                           unroll=4)(zero_body)
        plsc.parallel_loop(0, _ROWS_PER_BLOCK, unroll=8)(row_body)
        pltpu.sync_copy(
            hist,
            out_hbm.at[pl.ds(blk_row * _NBINS,
                             _ROWS_PER_BLOCK * _NBINS)])
        return 0

    lax.fori_loop(0, n_blocks, blk_body, 0)


def kernel(x):
    B, T, F = x.shape
    rows = B * T
    x_flat = x.reshape(rows * F)
    bnd = jnp.linspace(-3.0, 3.0, _NBINS - 1).astype(jnp.float32)
    bnd = jnp.concatenate([bnd, jnp.full((1,), 3.0, jnp.float32)])

    mesh = plsc.VectorSubcoreMesh(core_axis_name="c", subcore_axis_name="s")
    run = functools.partial(
        pl.kernel,
        out_type=jax.ShapeDtypeStruct((rows * _NBINS,), jnp.float32),
        mesh=mesh,
        compiler_params=pltpu.CompilerParams(needs_layout_passes=False),
        scratch_types=[
            pltpu.VMEM((_ROWS_PER_BLOCK * _FEATS,), jnp.float32),
            pltpu.VMEM((_ROWS_PER_BLOCK * _NBINS,), jnp.float32),
            pltpu.VMEM((_NBINS,), jnp.float32),
        ],
    )(_tokenizer_body)
    out = run(x_flat, bnd)
    return out.reshape(B, T, _NBINS)


# speedup vs baseline: 1.1315x; 1.1315x over previous
import functools

import jax
import jax.numpy as jnp
from jax import lax
from jax.experimental import pallas as pl
from jax.experimental.pallas import tpu as pltpu
from jax.experimental.pallas import tpu_sc as plsc

_NBINS = 32
_FEATS = 128
_LANES = 16
_NWORKERS = 32
_ROWS_PER_BLOCK = 256
_VECS_PER_ROW = _FEATS // _LANES


def _tokenizer_body(x_hbm, bnd_hbm, out_hbm, xbuf0, xbuf1, hist, bnd,
                    sem0, sem1):
    rows_total = x_hbm.shape[0] // _FEATS
    rows_per_worker = rows_total // _NWORKERS
    n_blocks = rows_per_worker // _ROWS_PER_BLOCK
    blk_elems = _ROWS_PER_BLOCK * _FEATS

    wid = lax.axis_index("c") * 16 + lax.axis_index("s")
    row0 = wid * rows_per_worker

    pltpu.sync_copy(bnd_hbm, bnd)

    val16 = jnp.full((_LANES,), 2.0 ** -7, jnp.float32)
    zeros16 = jnp.zeros((_LANES,), jnp.float32)

    def zero_body(i):
        hist[pl.ds(i * _LANES, _LANES)] = zeros16

    def make_row_body(xbuf):
        def row_body(r):
            rbase = r * _FEATS
            obase = r * _NBINS
            for j in range(_VECS_PER_ROW):
                x16 = xbuf[pl.ds(rbase + j * _LANES, _LANES)]
                t = x16 * jnp.float32(5.0) + jnp.float32(15.5)
                t = jnp.minimum(jnp.maximum(t, jnp.float32(0.0)),
                                jnp.float32(30.0))
                g = t.astype(jnp.int32)
                bg = plsc.load_gather(bnd, [g])
                y = g + (x16 >= bg).astype(jnp.int32)
                plsc.addupdate_scatter(hist, [y + obase], val16)
        return row_body

    row_body0 = make_row_body(xbuf0)
    row_body1 = make_row_body(xbuf1)

    def in_slice(blk):
        return x_hbm.at[pl.ds((row0 + blk * _ROWS_PER_BLOCK) * _FEATS,
                              blk_elems)]

    def compute_and_store(blk, row_body):
        plsc.parallel_loop(0, _ROWS_PER_BLOCK * _NBINS // _LANES,
                           unroll=4)(zero_body)
        plsc.parallel_loop(0, _ROWS_PER_BLOCK, unroll=4)(row_body)
        pltpu.sync_copy(
            hist,
            out_hbm.at[pl.ds((row0 + blk * _ROWS_PER_BLOCK) * _NBINS,
                             _ROWS_PER_BLOCK * _NBINS)])

    pltpu.async_copy(in_slice(0), xbuf0, sem0)

    def pair_body(p, _):
        k0 = p * 2
        pltpu.async_copy(in_slice(k0 + 1), xbuf1, sem1)
        pltpu.make_async_copy(in_slice(k0), xbuf0, sem0).wait()
        compute_and_store(k0, row_body0)
        nxt = jnp.minimum(k0 + 2, n_blocks - 1)
        pltpu.async_copy(in_slice(nxt), xbuf0, sem0)
        pltpu.make_async_copy(in_slice(k0 + 1), xbuf1, sem1).wait()
        compute_and_store(k0 + 1, row_body1)
        return 0

    lax.fori_loop(0, n_blocks // 2, pair_body, 0)
    pltpu.make_async_copy(in_slice(n_blocks - 1), xbuf0, sem0).wait()


def kernel(x):
    B, T, F = x.shape
    rows = B * T
    x_flat = x.reshape(rows * F)
    bnd = jnp.linspace(-3.0, 3.0, _NBINS - 1).astype(jnp.float32)
    bnd = jnp.concatenate([bnd, jnp.full((1,), 3.0, jnp.float32)])

    mesh = plsc.VectorSubcoreMesh(core_axis_name="c", subcore_axis_name="s")
    run = functools.partial(
        pl.kernel,
        out_type=jax.ShapeDtypeStruct((rows * _NBINS,), jnp.float32),
        mesh=mesh,
        compiler_params=pltpu.CompilerParams(needs_layout_passes=False),
        scratch_types=[
            pltpu.VMEM((_ROWS_PER_BLOCK * _FEATS,), jnp.float32),
            pltpu.VMEM((_ROWS_PER_BLOCK * _FEATS,), jnp.float32),
            pltpu.VMEM((_ROWS_PER_BLOCK * _NBINS,), jnp.float32),
            pltpu.VMEM((_NBINS,), jnp.float32),
            pltpu.SemaphoreType.DMA,
            pltpu.SemaphoreType.DMA,
        ],
    )(_tokenizer_body)
    out = run(x_flat, bnd)
    return out.reshape(B, T, _NBINS)


# speedup vs baseline: 1.1354x; 1.0035x over previous
import functools

import jax
import jax.numpy as jnp
from jax import lax
from jax.experimental import pallas as pl
from jax.experimental.pallas import tpu as pltpu
from jax.experimental.pallas import tpu_sc as plsc

_NBINS = 32
_FEATS = 128
_LANES = 16
_NWORKERS = 32
_ROWS_PER_BLOCK = 256
_VECS_PER_ROW = _FEATS // _LANES


def _tokenizer_body(x_hbm, bnd_hbm, out_hbm, xbuf0, xbuf1, hist, bnd,
                    sem0, sem1):
    rows_total = x_hbm.shape[0] // _FEATS
    rows_per_worker = rows_total // _NWORKERS
    n_blocks = rows_per_worker // _ROWS_PER_BLOCK
    blk_elems = _ROWS_PER_BLOCK * _FEATS

    wid = lax.axis_index("c") * 16 + lax.axis_index("s")
    row0 = wid * rows_per_worker

    pltpu.sync_copy(bnd_hbm, bnd)

    val16 = jnp.full((_LANES,), 2.0 ** -7, jnp.float32)
    zeros16 = jnp.zeros((_LANES,), jnp.float32)

    def zero_body(i):
        hist[pl.ds(i * _LANES, _LANES)] = zeros16

    def make_row_body(xbuf):
        def row_body(r):
            rbase = r * _FEATS
            obase = r * _NBINS
            for j in range(_VECS_PER_ROW):
                x16 = xbuf[pl.ds(rbase + j * _LANES, _LANES)]
                t = x16 * jnp.float32(5.0) + jnp.float32(15.5)
                t = jnp.minimum(jnp.maximum(t, jnp.float32(0.0)),
                                jnp.float32(30.0))
                g = t.astype(jnp.int32)
                bg = plsc.load_gather(bnd, [g])
                y = g + (x16 >= bg).astype(jnp.int32)
                plsc.addupdate_scatter(hist, [y + obase], val16)
        return row_body

    row_body0 = make_row_body(xbuf0)
    row_body1 = make_row_body(xbuf1)

    def in_slice(blk):
        return x_hbm.at[pl.ds((row0 + blk * _ROWS_PER_BLOCK) * _FEATS,
                              blk_elems)]

    def zero_hist():
        plsc.parallel_loop(0, _ROWS_PER_BLOCK * _NBINS // _LANES,
                           unroll=4)(zero_body)

    def compute_and_store(blk, row_body):
        plsc.parallel_loop(0, _ROWS_PER_BLOCK, unroll=4)(row_body)
        pltpu.sync_copy(
            hist,
            out_hbm.at[pl.ds((row0 + blk * _ROWS_PER_BLOCK) * _NBINS,
                             _ROWS_PER_BLOCK * _NBINS)])

    pltpu.async_copy(in_slice(0), xbuf0, sem0)

    def pair_body(p, _):
        k0 = p * 2
        pltpu.async_copy(in_slice(k0 + 1), xbuf1, sem1)
        zero_hist()
        pltpu.make_async_copy(in_slice(k0), xbuf0, sem0).wait()
        compute_and_store(k0, row_body0)
        nxt = jnp.minimum(k0 + 2, n_blocks - 1)
        pltpu.async_copy(in_slice(nxt), xbuf0, sem0)
        zero_hist()
        pltpu.make_async_copy(in_slice(k0 + 1), xbuf1, sem1).wait()
        compute_and_store(k0 + 1, row_body1)
        return 0

    lax.fori_loop(0, n_blocks // 2, pair_body, 0)
    pltpu.make_async_copy(in_slice(n_blocks - 1), xbuf0, sem0).wait()


def kernel(x):
    B, T, F = x.shape
    rows = B * T
    x_flat = x.reshape(rows * F)
    bnd = jnp.linspace(-3.0, 3.0, _NBINS - 1).astype(jnp.float32)
    bnd = jnp.concatenate([bnd, jnp.full((1,), 3.0, jnp.float32)])

    mesh = plsc.VectorSubcoreMesh(core_axis_name="c", subcore_axis_name="s")
    run = functools.partial(
        pl.kernel,
        out_type=jax.ShapeDtypeStruct((rows * _NBINS,), jnp.float32),
        mesh=mesh,
        compiler_params=pltpu.CompilerParams(needs_layout_passes=False),
        scratch_types=[
            pltpu.VMEM((_ROWS_PER_BLOCK * _FEATS,), jnp.float32),
            pltpu.VMEM((_ROWS_PER_BLOCK * _FEATS,), jnp.float32),
            pltpu.VMEM((_ROWS_PER_BLOCK * _NBINS,), jnp.float32),
            pltpu.VMEM((_NBINS,), jnp.float32),
            pltpu.SemaphoreType.DMA,
            pltpu.SemaphoreType.DMA,
        ],
    )(_tokenizer_body)
    out = run(x_flat, bnd)
    return out.reshape(B, T, _NBINS)


# speedup vs baseline: 1.2407x; 1.0927x over previous
import functools

import jax
import jax.numpy as jnp
from jax import lax
from jax.experimental import pallas as pl
from jax.experimental.pallas import tpu as pltpu
from jax.experimental.pallas import tpu_sc as plsc

_NBINS = 32
_FEATS = 128
_LANES = 16
_NWORKERS = 32
_ROWS_PER_BLOCK = 256
_VECS_PER_ROW = _FEATS // _LANES


def _tokenizer_body(x_hbm, bnd_hbm, out_hbm, xbuf0, xbuf1, hist, bnd,
                    sem0, sem1):
    rows_total = x_hbm.shape[0] // _FEATS
    rows_per_worker = rows_total // _NWORKERS
    n_blocks = rows_per_worker // _ROWS_PER_BLOCK
    blk_elems = _ROWS_PER_BLOCK * _FEATS

    wid = lax.axis_index("c") * 16 + lax.axis_index("s")
    row0 = wid * rows_per_worker

    pltpu.sync_copy(bnd_hbm, bnd)

    val16 = jnp.full((_LANES,), 2.0 ** -7, jnp.float32)
    zeros16 = jnp.zeros((_LANES,), jnp.float32)

    def zero_body(i):
        hist[pl.ds(i * _LANES, _LANES)] = zeros16

    def make_row_body(xbuf):
        def row_body(r):
            rbase = r * _FEATS
            obase = r * _NBINS
            for j in range(_VECS_PER_ROW):
                x16 = xbuf[pl.ds(rbase + j * _LANES, _LANES)]
                t = x16 * jnp.float32(5.0) + jnp.float32(15.5)
                t = jnp.minimum(jnp.maximum(t, jnp.float32(0.0)),
                                jnp.float32(30.0))
                g = t.astype(jnp.int32)
                bg = plsc.load_gather(bnd, [g])
                y = g + (x16 >= bg).astype(jnp.int32)
                cnt, last = plsc.scan_count(y)
                val = cnt.astype(jnp.float32) * jnp.float32(2.0 ** -7)
                plsc.addupdate_scatter(hist, [y + obase], val, mask=last)
        return row_body

    row_body0 = make_row_body(xbuf0)
    row_body1 = make_row_body(xbuf1)

    def in_slice(blk):
        return x_hbm.at[pl.ds((row0 + blk * _ROWS_PER_BLOCK) * _FEATS,
                              blk_elems)]

    def zero_hist():
        plsc.parallel_loop(0, _ROWS_PER_BLOCK * _NBINS // _LANES,
                           unroll=4)(zero_body)

    def compute_and_store(blk, row_body):
        plsc.parallel_loop(0, _ROWS_PER_BLOCK, unroll=4)(row_body)
        pltpu.sync_copy(
            hist,
            out_hbm.at[pl.ds((row0 + blk * _ROWS_PER_BLOCK) * _NBINS,
                             _ROWS_PER_BLOCK * _NBINS)])

    pltpu.async_copy(in_slice(0), xbuf0, sem0)

    def pair_body(p, _):
        k0 = p * 2
        pltpu.async_copy(in_slice(k0 + 1), xbuf1, sem1)
        zero_hist()
        pltpu.make_async_copy(in_slice(k0), xbuf0, sem0).wait()
        compute_and_store(k0, row_body0)
        nxt = jnp.minimum(k0 + 2, n_blocks - 1)
        pltpu.async_copy(in_slice(nxt), xbuf0, sem0)
        zero_hist()
        pltpu.make_async_copy(in_slice(k0 + 1), xbuf1, sem1).wait()
        compute_and_store(k0 + 1, row_body1)
        return 0

    lax.fori_loop(0, n_blocks // 2, pair_body, 0)
    pltpu.make_async_copy(in_slice(n_blocks - 1), xbuf0, sem0).wait()


def kernel(x):
    B, T, F = x.shape
    rows = B * T
    x_flat = x.reshape(rows * F)
    bnd = jnp.linspace(-3.0, 3.0, _NBINS - 1).astype(jnp.float32)
    bnd = jnp.concatenate([bnd, jnp.full((1,), 3.0, jnp.float32)])

    mesh = plsc.VectorSubcoreMesh(core_axis_name="c", subcore_axis_name="s")
    run = functools.partial(
        pl.kernel,
        out_type=jax.ShapeDtypeStruct((rows * _NBINS,), jnp.float32),
        mesh=mesh,
        compiler_params=pltpu.CompilerParams(needs_layout_passes=False),
        scratch_types=[
            pltpu.VMEM((_ROWS_PER_BLOCK * _FEATS,), jnp.float32),
            pltpu.VMEM((_ROWS_PER_BLOCK * _FEATS,), jnp.float32),
            pltpu.VMEM((_ROWS_PER_BLOCK * _NBINS,), jnp.float32),
            pltpu.VMEM((_NBINS,), jnp.float32),
            pltpu.SemaphoreType.DMA,
            pltpu.SemaphoreType.DMA,
        ],
    )(_tokenizer_body)
    out = run(x_flat, bnd)
    return out.reshape(B, T, _NBINS)


# speedup vs baseline: 1.2437x; 1.0024x over previous
import functools

import jax
import jax.numpy as jnp
from jax import lax
from jax.experimental import pallas as pl
from jax.experimental.pallas import tpu as pltpu
from jax.experimental.pallas import tpu_sc as plsc

_NBINS = 32
_FEATS = 128
_LANES = 16
_NWORKERS = 32
_ROWS_PER_BLOCK = 256
_VECS_PER_ROW = _FEATS // _LANES


def _tokenizer_body(x_hbm, bnd_hbm, out_hbm, xbuf0, xbuf1, hist0, hist1,
                    bnd, sem0, sem1, osem0, osem1):
    rows_total = x_hbm.shape[0] // _FEATS
    rows_per_worker = rows_total // _NWORKERS
    n_blocks = rows_per_worker // _ROWS_PER_BLOCK
    blk_elems = _ROWS_PER_BLOCK * _FEATS

    wid = lax.axis_index("c") * 16 + lax.axis_index("s")
    row0 = wid * rows_per_worker

    pltpu.sync_copy(bnd_hbm, bnd)

    zeros16 = jnp.zeros((_LANES,), jnp.float32)

    def make_zero_body(hist):
        def zero_body(i):
            hist[pl.ds(i * _LANES, _LANES)] = zeros16
        return zero_body

    def make_row_body(xbuf, hist):
        def row_body(r):
            rbase = r * _FEATS
            obase = r * _NBINS
            for j in range(_VECS_PER_ROW):
                x16 = xbuf[pl.ds(rbase + j * _LANES, _LANES)]
                t = x16 * jnp.float32(5.0) + jnp.float32(15.5)
                t = jnp.minimum(jnp.maximum(t, jnp.float32(0.0)),
                                jnp.float32(30.0))
                g = t.astype(jnp.int32)
                bg = plsc.load_gather(bnd, [g])
                y = g + (x16 >= bg).astype(jnp.int32)
                cnt, last = plsc.scan_count(y)
                val = cnt.astype(jnp.float32) * jnp.float32(2.0 ** -7)
                plsc.addupdate_scatter(hist, [y + obase], val, mask=last)
        return row_body

    def in_slice(blk):
        return x_hbm.at[pl.ds((row0 + blk * _ROWS_PER_BLOCK) * _FEATS,
                              blk_elems)]

    def out_slice(blk):
        return out_hbm.at[pl.ds((row0 + blk * _ROWS_PER_BLOCK) * _NBINS,
                                _ROWS_PER_BLOCK * _NBINS)]

    def zero_hist(hist):
        plsc.parallel_loop(0, _ROWS_PER_BLOCK * _NBINS // _LANES,
                           unroll=4)(make_zero_body(hist))

    def compute(xbuf, hist):
        plsc.parallel_loop(0, _ROWS_PER_BLOCK, unroll=4)(
            make_row_body(xbuf, hist))

    pltpu.async_copy(in_slice(0), xbuf0, sem0)
    pltpu.async_copy(in_slice(1), xbuf1, sem1)
    zero_hist(hist0)
    pltpu.make_async_copy(in_slice(0), xbuf0, sem0).wait()
    compute(xbuf0, hist0)
    pltpu.async_copy(hist0, out_slice(0), osem0)
    pltpu.async_copy(in_slice(2), xbuf0, sem0)
    zero_hist(hist1)
    pltpu.make_async_copy(in_slice(1), xbuf1, sem1).wait()
    compute(xbuf1, hist1)
    pltpu.async_copy(hist1, out_slice(1), osem1)

    def pair_body(p, _):
        k0 = p * 2
        pltpu.async_copy(in_slice(k0 + 1), xbuf1, sem1)
        pltpu.make_async_copy(hist0, out_slice(k0 - 2), osem0).wait()
        zero_hist(hist0)
        pltpu.make_async_copy(in_slice(k0), xbuf0, sem0).wait()
        compute(xbuf0, hist0)
        pltpu.async_copy(hist0, out_slice(k0), osem0)
        nxt = jnp.minimum(k0 + 2, n_blocks - 1)
        pltpu.async_copy(in_slice(nxt), xbuf0, sem0)
        pltpu.make_async_copy(hist1, out_slice(k0 - 1), osem1).wait()
        zero_hist(hist1)
        pltpu.make_async_copy(in_slice(k0 + 1), xbuf1, sem1).wait()
        compute(xbuf1, hist1)
        pltpu.async_copy(hist1, out_slice(k0 + 1), osem1)
        return 0

    lax.fori_loop(1, n_blocks // 2, pair_body, 0)
    pltpu.make_async_copy(in_slice(n_blocks - 1), xbuf0, sem0).wait()
    pltpu.make_async_copy(hist0, out_slice(n_blocks - 2), osem0).wait()
    pltpu.make_async_copy(hist1, out_slice(n_blocks - 1), osem1).wait()


def kernel(x):
    B, T, F = x.shape
    rows = B * T
    x_flat = x.reshape(rows * F)
    bnd = jnp.linspace(-3.0, 3.0, _NBINS - 1).astype(jnp.float32)
    bnd = jnp.concatenate([bnd, jnp.full((1,), 3.0, jnp.float32)])

    mesh = plsc.VectorSubcoreMesh(core_axis_name="c", subcore_axis_name="s")
    run = functools.partial(
        pl.kernel,
        out_type=jax.ShapeDtypeStruct((rows * _NBINS,), jnp.float32),
        mesh=mesh,
        compiler_params=pltpu.CompilerParams(needs_layout_passes=False),
        scratch_types=[
            pltpu.VMEM((_ROWS_PER_BLOCK * _FEATS,), jnp.float32),
            pltpu.VMEM((_ROWS_PER_BLOCK * _FEATS,), jnp.float32),
            pltpu.VMEM((_ROWS_PER_BLOCK * _NBINS,), jnp.float32),
            pltpu.VMEM((_ROWS_PER_BLOCK * _NBINS,), jnp.float32),
            pltpu.VMEM((_NBINS,), jnp.float32),
            pltpu.SemaphoreType.DMA,
            pltpu.SemaphoreType.DMA,
            pltpu.SemaphoreType.DMA,
            pltpu.SemaphoreType.DMA,
        ],
    )(_tokenizer_body)
    out = run(x_flat, bnd)
    return out.reshape(B, T, _NBINS)


# speedup vs baseline: 1.3134x; 1.0560x over previous
import functools

import jax
import jax.numpy as jnp
from jax import lax
from jax.experimental import pallas as pl
from jax.experimental.pallas import tpu as pltpu
from jax.experimental.pallas import tpu_sc as plsc

_NBINS = 32
_FEATS = 128
_LANES = 16
_NWORKERS = 32
_ROWS_PER_BLOCK = 256
_VECS_PER_ROW = _FEATS // _LANES


def _tokenizer_body(x_hbm, bnd_hbm, out_hbm, xbuf0, xbuf1, hist0, hist1,
                    bnd, sem0, sem1, osem0, osem1):
    rows_total = x_hbm.shape[0] // _FEATS
    rows_per_worker = rows_total // _NWORKERS
    n_blocks = rows_per_worker // _ROWS_PER_BLOCK
    blk_elems = _ROWS_PER_BLOCK * _FEATS

    wid = lax.axis_index("c") * 16 + lax.axis_index("s")
    row0 = wid * rows_per_worker

    pltpu.sync_copy(bnd_hbm, bnd)

    zeros16 = jnp.zeros((_LANES,), jnp.float32)

    def make_row_body(xbuf, hist):
        def row_body(r):
            rbase = r * _FEATS
            obase = r * _NBINS
            hrow = hist.at[pl.ds(obase, _NBINS)]
            hrow[pl.ds(0, _LANES)] = zeros16
            hrow[pl.ds(_LANES, _LANES)] = zeros16
            for j in range(_VECS_PER_ROW):
                x16 = xbuf[pl.ds(rbase + j * _LANES, _LANES)]
                t = x16 * jnp.float32(5.0) + jnp.float32(15.5)
                t = jnp.minimum(jnp.maximum(t, jnp.float32(0.0)),
                                jnp.float32(30.0))
                g = t.astype(jnp.int32)
                bg = plsc.load_gather(bnd, [g])
                y = g + (x16 >= bg).astype(jnp.int32)
                cnt, last = plsc.scan_count(y)
                val = cnt.astype(jnp.float32) * jnp.float32(2.0 ** -7)
                plsc.addupdate_scatter(hrow, [y], val, mask=last)
        return row_body

    def in_slice(blk):
        return x_hbm.at[pl.ds((row0 + blk * _ROWS_PER_BLOCK) * _FEATS,
                              blk_elems)]

    def out_slice(blk):
        return out_hbm.at[pl.ds((row0 + blk * _ROWS_PER_BLOCK) * _NBINS,
                                _ROWS_PER_BLOCK * _NBINS)]

    def compute(xbuf, hist):
        plsc.parallel_loop(0, _ROWS_PER_BLOCK, unroll=4)(
            make_row_body(xbuf, hist))

    pltpu.async_copy(in_slice(0), xbuf0, sem0)
    pltpu.async_copy(in_slice(1), xbuf1, sem1)
    pltpu.make_async_copy(in_slice(0), xbuf0, sem0).wait()
    compute(xbuf0, hist0)
    pltpu.async_copy(hist0, out_slice(0), osem0)
    pltpu.async_copy(in_slice(2), xbuf0, sem0)
    pltpu.make_async_copy(in_slice(1), xbuf1, sem1).wait()
    compute(xbuf1, hist1)
    pltpu.async_copy(hist1, out_slice(1), osem1)

    def pair_body(p, _):
        k0 = p * 2
        pltpu.async_copy(in_slice(k0 + 1), xbuf1, sem1)
        pltpu.make_async_copy(hist0, out_slice(k0 - 2), osem0).wait()
        pltpu.make_async_copy(in_slice(k0), xbuf0, sem0).wait()
        compute(xbuf0, hist0)
        pltpu.async_copy(hist0, out_slice(k0), osem0)
        nxt = jnp.minimum(k0 + 2, n_blocks - 1)
        pltpu.async_copy(in_slice(nxt), xbuf0, sem0)
        pltpu.make_async_copy(hist1, out_slice(k0 - 1), osem1).wait()
        pltpu.make_async_copy(in_slice(k0 + 1), xbuf1, sem1).wait()
        compute(xbuf1, hist1)
        pltpu.async_copy(hist1, out_slice(k0 + 1), osem1)
        return 0

    lax.fori_loop(1, n_blocks // 2, pair_body, 0)
    pltpu.make_async_copy(in_slice(n_blocks - 1), xbuf0, sem0).wait()
    pltpu.make_async_copy(hist0, out_slice(n_blocks - 2), osem0).wait()
    pltpu.make_async_copy(hist1, out_slice(n_blocks - 1), osem1).wait()


def kernel(x):
    B, T, F = x.shape
    rows = B * T
    x_flat = x.reshape(rows * F)
    bnd = jnp.linspace(-3.0, 3.0, _NBINS - 1).astype(jnp.float32)
    bnd = jnp.concatenate([bnd, jnp.full((1,), 3.0, jnp.float32)])

    mesh = plsc.VectorSubcoreMesh(core_axis_name="c", subcore_axis_name="s")
    run = functools.partial(
        pl.kernel,
        out_type=jax.ShapeDtypeStruct((rows * _NBINS,), jnp.float32),
        mesh=mesh,
        compiler_params=pltpu.CompilerParams(needs_layout_passes=False),
        scratch_types=[
            pltpu.VMEM((_ROWS_PER_BLOCK * _FEATS,), jnp.float32),
            pltpu.VMEM((_ROWS_PER_BLOCK * _FEATS,), jnp.float32),
            pltpu.VMEM((_ROWS_PER_BLOCK * _NBINS,), jnp.float32),
            pltpu.VMEM((_ROWS_PER_BLOCK * _NBINS,), jnp.float32),
            pltpu.VMEM((_NBINS,), jnp.float32),
            pltpu.SemaphoreType.DMA,
            pltpu.SemaphoreType.DMA,
            pltpu.SemaphoreType.DMA,
            pltpu.SemaphoreType.DMA,
        ],
    )(_tokenizer_body)
    out = run(x_flat, bnd)
    return out.reshape(B, T, _NBINS)
